# baseline (speedup 1.0000x reference)
import functools

import jax
import jax.numpy as jnp
from jax import lax
from jax.experimental import pallas as pl
from jax.experimental.pallas import tpu as pltpu
from jax.experimental.pallas import tpu_sc as plsc

_C = 8192
_D = 32
_M_TILE = 256
_HALF = _C // 2


def _tree_sumsq(x):
    a = x * x
    v = ((a[..., 0:8] + a[..., 8:16]) + a[..., 16:24]) + a[..., 24:32]
    v = v[..., :4] + v[..., 4:]
    v = v[..., :2] + v[..., 2:]
    return v[..., 0] + v[..., 1]


def _argmin_body(z_ref, e_ref, code_ref, msum_ref):
    z = z_ref[...]
    e = e_ref[...]
    z2 = _tree_sumsq(z)[:, None]
    e2 = _tree_sumsq(e)[None, :]
    lhs = (2.0 * z).astype(jnp.bfloat16)
    e_hi = e.astype(jnp.bfloat16)
    dn = (((1,), (1,)), ((), ()))
    mm = lax.dot_general(lhs, e_hi, dn, preferred_element_type=jnp.float32)
    dist = (z2 - mm) + e2

    d1 = dist[:, :_HALF]
    d2 = dist[:, _HALF:]
    m1 = jnp.min(d1, axis=-1, keepdims=True)
    m2 = jnp.min(d2, axis=-1, keepdims=True)
    iota = lax.broadcasted_iota(jnp.int32, (_M_TILE, _HALF), 1)
    i1 = jnp.min(jnp.where(d1 == m1, iota, _C), axis=-1)
    i2 = jnp.min(jnp.where(d2 == m2, iota, _C), axis=-1) + _HALF

    m1r = m1.astype(jnp.bfloat16).astype(jnp.float32)
    take2 = (m2 < m1r)[:, 0]
    code = jnp.where(take2, i2, i1)
    msel = jnp.where(take2, m2[:, 0], m1[:, 0])

    code_ref[0, 0, :] = code

    @pl.when(pl.program_id(0) == 0)
    def _():
        msum_ref[...] = jnp.zeros((1, 1), jnp.float32)

    msum_ref[...] += jnp.sum(msel).reshape(1, 1)


def _argmin_call(flat, embedding):
    b = flat.shape[0]
    nt = b // _M_TILE
    code3, msum = pl.pallas_call(
        _argmin_body,
        grid=(nt,),
        in_specs=[
            pl.BlockSpec((_M_TILE, _D), lambda i: (i, 0)),
            pl.BlockSpec((_C, _D), lambda i: (0, 0)),
        ],
        out_specs=[
            pl.BlockSpec((1, 1, _M_TILE), lambda i: (i, 0, 0)),
            pl.BlockSpec((1, 1), lambda i: (0, 0)),
        ],
        out_shape=[
            jax.ShapeDtypeStruct((nt, 1, _M_TILE), jnp.int32),
            jax.ShapeDtypeStruct((1, 1), jnp.float32),
        ],
    )(flat, embedding)
    return code3.reshape(b), msum[0, 0]


_ROW = 128


@functools.cache
def _make_gather(b):
    info = plsc.get_sparse_core_info()
    nc, ns = info.num_cores, info.num_subcores
    nw = nc * ns
    b_per_w = b // nw
    ch = 128
    n_ch = b_per_w // ch
    mesh = plsc.VectorSubcoreMesh(core_axis_name="c", subcore_axis_name="s")

    @functools.partial(
        pl.kernel,
        mesh=mesh,
        out_type=jax.ShapeDtypeStruct((b, _ROW), jnp.float32),
        scratch_types=[
            pltpu.VMEM((n_ch, ch), jnp.int32),
            pltpu.VMEM((b_per_w, _ROW), jnp.float32),
            pltpu.SemaphoreType.DMA,
        ],
    )
    def gather_k(table_hbm, idx_hbm, out_hbm, idx_v, rows_v, sem):
        wid = lax.axis_index("s") * nc + lax.axis_index("c")
        base = wid * b_per_w
        for j in range(n_ch):
            pltpu.sync_copy(idx_hbm.at[pl.ds(base + j * ch, ch)], idx_v.at[j])
            pltpu.async_copy(table_hbm.at[idx_v.at[j]],
                             rows_v.at[pl.ds(j * ch, ch)], sem).wait()
        pltpu.sync_copy(rows_v, out_hbm.at[pl.ds(base, b_per_w)])

    return gather_k


def kernel(z_e, embedding):
    b = z_e.shape[0] * z_e.shape[1]
    flat = z_e.reshape(b, _D)
    code_flat, msum = _argmin_call(flat, embedding)
    table = jnp.pad(embedding, ((0, 0), (0, _ROW - _D)))
    z_q = _make_gather(b)(table, code_flat)[:, :_D].reshape(z_e.shape)
    z_q_st = z_e + (z_q - z_e)
    commitment = (msum / (b * _D)) * 0.25
    return (z_q_st, code_flat.reshape(z_e.shape[:-1]), commitment)


# speedup vs baseline: 1.5624x; 1.5624x over previous
import functools

import jax
import jax.numpy as jnp
from jax import lax
from jax.experimental import pallas as pl
from jax.experimental.pallas import tpu as pltpu
from jax.experimental.pallas import tpu_sc as plsc

_C = 8192
_D = 32
_M_TILE = 512
_HALF = _C // 2


def _tree_sumsq(x):
    a = x * x
    v = ((a[..., 0:8] + a[..., 8:16]) + a[..., 16:24]) + a[..., 24:32]
    v = v[..., :4] + v[..., 4:]
    v = v[..., :2] + v[..., 2:]
    return v[..., 0] + v[..., 1]


def _argmin_body(z_ref, e_ref, code_ref, msum_ref):
    z = z_ref[...]
    e = e_ref[...]
    z2 = _tree_sumsq(z)[:, None]
    e2 = _tree_sumsq(e)[None, :]
    lhs = (2.0 * z).astype(jnp.bfloat16)
    e_hi = e.astype(jnp.bfloat16)
    dn = (((1,), (1,)), ((), ()))
    mm = lax.dot_general(lhs, e_hi, dn, preferred_element_type=jnp.float32)
    dist = (z2 - mm) + e2

    d1 = dist[:, :_HALF]
    d2 = dist[:, _HALF:]
    m1 = jnp.min(d1, axis=-1, keepdims=True)
    m2 = jnp.min(d2, axis=-1, keepdims=True)
    iota = lax.broadcasted_iota(jnp.int32, (_M_TILE, _HALF), 1)
    i1 = jnp.min(jnp.where(d1 == m1, iota, _C), axis=-1)
    i2 = jnp.min(jnp.where(d2 == m2, iota, _C), axis=-1) + _HALF

    m1r = m1.astype(jnp.bfloat16).astype(jnp.float32)
    take2 = (m2 < m1r)[:, 0]
    code = jnp.where(take2, i2, i1)
    msel = jnp.where(take2, m2[:, 0], m1[:, 0])

    code_ref[0, 0, :] = code

    @pl.when(pl.program_id(0) == 0)
    def _():
        msum_ref[...] = jnp.zeros((1, 1), jnp.float32)

    msum_ref[...] += jnp.sum(msel).reshape(1, 1)


def _argmin_call(flat, embedding):
    b = flat.shape[0]
    nt = b // _M_TILE
    code3, msum = pl.pallas_call(
        _argmin_body,
        grid=(nt,),
        in_specs=[
            pl.BlockSpec((_M_TILE, _D), lambda i: (i, 0)),
            pl.BlockSpec((_C, _D), lambda i: (0, 0)),
        ],
        out_specs=[
            pl.BlockSpec((1, 1, _M_TILE), lambda i: (i, 0, 0)),
            pl.BlockSpec((1, 1), lambda i: (0, 0)),
        ],
        out_shape=[
            jax.ShapeDtypeStruct((nt, 1, _M_TILE), jnp.int32),
            jax.ShapeDtypeStruct((1, 1), jnp.float32),
        ],
    )(flat, embedding)
    return code3.reshape(b), msum[0, 0]


_ROW = 128


@functools.cache
def _make_gather(b):
    info = plsc.get_sparse_core_info()
    nc, ns = info.num_cores, info.num_subcores
    nw = nc * ns
    b_per_w = b // nw
    ch = 128
    n_ch = b_per_w // ch
    mesh = plsc.VectorSubcoreMesh(core_axis_name="c", subcore_axis_name="s")

    @functools.partial(
        pl.kernel,
        mesh=mesh,
        out_type=jax.ShapeDtypeStruct((b, _ROW), jnp.float32),
        scratch_types=[
            pltpu.VMEM((n_ch, ch), jnp.int32),
            pltpu.VMEM((b_per_w, _ROW), jnp.float32),
            pltpu.SemaphoreType.DMA,
        ],
    )
    def gather_k(table_hbm, idx_hbm, out_hbm, idx_v, rows_v, sem):
        wid = lax.axis_index("s") * nc + lax.axis_index("c")
        base = wid * b_per_w
        for j in range(n_ch):
            pltpu.sync_copy(idx_hbm.at[pl.ds(base + j * ch, ch)], idx_v.at[j])
            pltpu.async_copy(table_hbm.at[idx_v.at[j]],
                             rows_v.at[pl.ds(j * ch, ch)], sem).wait()
        pltpu.sync_copy(rows_v, out_hbm.at[pl.ds(base, b_per_w)])

    return gather_k


def kernel(z_e, embedding):
    b = z_e.shape[0] * z_e.shape[1]
    flat = z_e.reshape(b, _D)
    code_flat, msum = _argmin_call(flat, embedding)
    table = jnp.pad(embedding, ((0, 0), (0, _ROW - _D)))
    z_q = _make_gather(b)(table, code_flat)[:, :_D].reshape(z_e.shape)
    z_q_st = z_e + (z_q - z_e)
    commitment = (msum / (b * _D)) * 0.25
    return (z_q_st, code_flat.reshape(z_e.shape[:-1]), commitment)


# speedup vs baseline: 2.1436x; 1.3720x over previous
import functools

import jax
import jax.numpy as jnp
from jax import lax
from jax.experimental import pallas as pl
from jax.experimental.pallas import tpu as pltpu
from jax.experimental.pallas import tpu_sc as plsc

_C = 8192
_D = 32
_M_TILE = 1024
_HALF = _C // 2


def _tree_sumsq(x):
    a = x * x
    v = ((a[..., 0:8] + a[..., 8:16]) + a[..., 16:24]) + a[..., 24:32]
    v = v[..., :4] + v[..., 4:]
    v = v[..., :2] + v[..., 2:]
    return v[..., 0] + v[..., 1]


def _argmin_body(z_ref, e_ref, code_ref, msum_ref):
    z = z_ref[...]
    e = e_ref[...]
    z2 = _tree_sumsq(z)[:, None]
    e2 = _tree_sumsq(e)[None, :]
    lhs = (2.0 * z).astype(jnp.bfloat16)
    e_hi = e.astype(jnp.bfloat16)
    dn = (((1,), (1,)), ((), ()))
    mm = lax.dot_general(lhs, e_hi, dn, preferred_element_type=jnp.float32)
    dist = (z2 - mm) + e2

    d1 = dist[:, :_HALF]
    d2 = dist[:, _HALF:]
    m1 = jnp.min(d1, axis=-1, keepdims=True)
    m2 = jnp.min(d2, axis=-1, keepdims=True)
    iota = lax.broadcasted_iota(jnp.int32, (_M_TILE, _HALF), 1)
    i1 = jnp.min(jnp.where(d1 == m1, iota, _C), axis=-1)
    i2 = jnp.min(jnp.where(d2 == m2, iota, _C), axis=-1) + _HALF

    m1r = m1.astype(jnp.bfloat16).astype(jnp.float32)
    take2 = (m2 < m1r)[:, 0]
    code = jnp.where(take2, i2, i1)
    msel = jnp.where(take2, m2[:, 0], m1[:, 0])

    code_ref[0, 0, :] = code

    @pl.when(pl.program_id(0) == 0)
    def _():
        msum_ref[...] = jnp.zeros((1, 1), jnp.float32)

    msum_ref[...] += jnp.sum(msel).reshape(1, 1)


def _argmin_call(flat, embedding):
    b = flat.shape[0]
    nt = b // _M_TILE
    code3, msum = pl.pallas_call(
        _argmin_body,
        grid=(nt,),
        in_specs=[
            pl.BlockSpec((_M_TILE, _D), lambda i: (i, 0)),
            pl.BlockSpec((_C, _D), lambda i: (0, 0)),
        ],
        out_specs=[
            pl.BlockSpec((1, 1, _M_TILE), lambda i: (i, 0, 0)),
            pl.BlockSpec((1, 1), lambda i: (0, 0)),
        ],
        out_shape=[
            jax.ShapeDtypeStruct((nt, 1, _M_TILE), jnp.int32),
            jax.ShapeDtypeStruct((1, 1), jnp.float32),
        ],
    )(flat, embedding)
    return code3.reshape(b), msum[0, 0]


_ROW = 128


@functools.cache
def _make_gather(b):
    info = plsc.get_sparse_core_info()
    nc, ns = info.num_cores, info.num_subcores
    nw = nc * ns
    b_per_w = b // nw
    ch = 128
    n_ch = b_per_w // ch
    mesh = plsc.VectorSubcoreMesh(core_axis_name="c", subcore_axis_name="s")

    @functools.partial(
        pl.kernel,
        mesh=mesh,
        out_type=jax.ShapeDtypeStruct((b, _ROW), jnp.float32),
        scratch_types=[
            pltpu.VMEM((n_ch, ch), jnp.int32),
            pltpu.VMEM((b_per_w, _ROW), jnp.float32),
            pltpu.SemaphoreType.DMA,
        ],
    )
    def gather_k(table_hbm, idx_hbm, out_hbm, idx_v, rows_v, sem):
        wid = lax.axis_index("s") * nc + lax.axis_index("c")
        base = wid * b_per_w
        for j in range(n_ch):
            pltpu.sync_copy(idx_hbm.at[pl.ds(base + j * ch, ch)], idx_v.at[j])
            pltpu.async_copy(table_hbm.at[idx_v.at[j]],
                             rows_v.at[pl.ds(j * ch, ch)], sem).wait()
        pltpu.sync_copy(rows_v, out_hbm.at[pl.ds(base, b_per_w)])

    return gather_k


def kernel(z_e, embedding):
    b = z_e.shape[0] * z_e.shape[1]
    flat = z_e.reshape(b, _D)
    code_flat, msum = _argmin_call(flat, embedding)
    table = jnp.pad(embedding, ((0, 0), (0, _ROW - _D)))
    z_q = _make_gather(b)(table, code_flat)[:, :_D].reshape(z_e.shape)
    z_q_st = z_e + (z_q - z_e)
    commitment = (msum / (b * _D)) * 0.25
    return (z_q_st, code_flat.reshape(z_e.shape[:-1]), commitment)
